# baseline (speedup 1.0000x reference)
import functools

import jax
import jax.numpy as jnp
from jax import lax
from jax.experimental import pallas as pl
from jax.experimental.pallas import tpu as pltpu
from jax.experimental.pallas import tpu_sc as plsc

F32 = jnp.float32
BF16 = jnp.bfloat16


def _mm(a, b):
    return lax.dot_general(a, b, (((1,), (0,)), ((), ())),
                           preferred_element_type=F32)


def _mm_t(a, b):
    return lax.dot_general(a, b, (((0,), (0,)), ((), ())),
                           preferred_element_type=F32)


def _smap(f, *ls):
    return [f(*xs) for xs in zip(*ls)]


def _sub_body(xp_ref, et_ref, ws0_ref, wd0_ref, wr0_ref, am0_ref,
              ws1_ref, wd1_ref, am1_ref, wg_ref, wl_ref, em_ref, bp_ref,
              out_ref):
    bp = bp_ref[...]
    b_src1 = bp[0:1, :]
    b_dst1 = bp[1:2, :]
    b_gate = bp[2:3, 0:1]
    b_lin = bp[3:4, 0:out_ref.shape[2]]
    blk = xp_ref.shape[0]
    n_nodes = xp_ref.shape[1]
    n_edges = et_ref.shape[1]
    ks = list(range(blk))

    xs = [xp_ref[k] for k in ks]
    n_iota = lax.broadcasted_iota(jnp.int32, (n_edges, n_nodes), 1)
    oh_src = [(et_ref[k][:, 0:1] == n_iota).astype(BF16) for k in ks]
    oh_dst = [(et_ref[k][:, 1:2] == n_iota).astype(BF16) for k in ks]

    def gat_layer(fss, fds, ress, am_ref):
        am = am_ref[...]
        fs_src = _smap(lambda o, f: _mm(o, f), oh_src, fss)
        fd_dst = _smap(lambda o, f: _mm(o, f), oh_dst, fds)
        e = _smap(lambda a, b: jnp.where(a + b >= 0, a + b, (a + b) * 0.2),
                  fs_src, fd_dst)
        eb = _smap(lambda x: x.astype(BF16), e)
        logits_t = _smap(
            lambda x: lax.dot_general(am, x, (((0,), (1,)), ((), ())),
                                      preferred_element_type=F32), eb)
        gmax = _smap(lambda l: jnp.max(l, axis=(0, 1), keepdims=True),
                     logits_t)
        exl_t = _smap(lambda l, m: jnp.exp(l - m).astype(BF16),
                      logits_t, gmax)
        denom_t = _smap(lambda x, o: _mm(x, o), exl_t, oh_dst)
        denom_dst_t = _smap(
            lambda d, o: lax.dot_general(d.astype(BF16), o,
                                         (((1,), (1,)), ((), ())),
                                         preferred_element_type=F32),
            denom_t, oh_dst)
        a_t = _smap(lambda x, d: (x / jnp.maximum(d, 1e-9)).astype(BF16),
                    exl_t, denom_dst_t)
        em = em_ref[...]
        a_exp = _smap(lambda a: _mm_t(a, em), a_t)
        wgt = _smap(lambda a, f: (a * f).astype(BF16), a_exp, fs_src)
        rst = _smap(lambda o, w: _mm_t(o, w), oh_dst, wgt)
        return _smap(lambda r, q: jnp.maximum(r + q, 0.0), rst, ress)

    def pool(hs):
        cmax = _smap(lambda h: jnp.max(h, axis=0, keepdims=True), hs)
        ex = _smap(lambda h, c: jnp.exp(h - c), hs, cmax)
        newh = _smap(
            lambda x: (x * (1.0 / jnp.sum(x, axis=0, keepdims=True))
                       ).astype(BF16), ex)
        wg = wg_ref[...]
        g_t = _smap(
            lambda nh: lax.dot_general(wg, nh, (((0,), (1,)), ((), ())),
                                       preferred_element_type=F32) + b_gate,
            newh)
        gmx = _smap(lambda g: jnp.max(g, axis=1, keepdims=True), g_t)
        gex = _smap(lambda g, m: jnp.exp(g - m), g_t, gmx)
        gate_t = _smap(
            lambda x: (x * (1.0 / jnp.sum(x, axis=1, keepdims=True))
                       ).astype(BF16), gex)
        return _smap(lambda g, nh: _mm(g, nh), gate_t, newh)

    ws0 = ws0_ref[...]
    wd0 = wd0_ref[...]
    wr0 = wr0_ref[...]
    fs0 = _smap(lambda x: _mm(x, ws0).astype(BF16), xs)
    fd0 = _smap(lambda x: _mm(x, wd0).astype(BF16), xs)
    res0 = _smap(lambda x: _mm(x, wr0), xs)
    h1 = gat_layer(fs0, fd0, res0, am0_ref)
    hg = pool(h1)
    h1b = _smap(lambda h: h.astype(BF16), h1)
    ws1 = ws1_ref[...]
    wd1 = wd1_ref[...]
    fs1 = _smap(lambda h: (_mm(h, ws1) + b_src1).astype(BF16), h1b)
    fd1 = _smap(lambda h: (_mm(h, wd1) + b_dst1).astype(BF16), h1b)
    h2 = gat_layer(fs1, fd1, h1, am1_ref)
    hg2 = pool(h2)
    wl = wl_ref[...]
    for k in ks:
        out_ref[k] = _mm((hg[k] + hg2[k]).astype(BF16), wl) + b_lin


def _sc_deg_body(src_hbm, dst_hbm, degs_out, idx_s, idx_d, zb, ob,
                 sh_dego, sh_degi, sem):
    cid = lax.axis_index("c")
    sid = lax.axis_index("s")
    for i in range(32):
        for c in range(zb.shape[1] // 16):
            zb[i, pl.ds(16 * c, 16)] = jnp.zeros((16,), F32)
            ob[i, pl.ds(16 * c, 16)] = jnp.ones((16,), F32)
    pltpu.sync_copy(zb, sh_dego.at[pl.ds(sid * 32, 32)])
    pltpu.sync_copy(zb, sh_degi.at[pl.ds(sid * 32, 32)])
    plsc.subcore_barrier()
    rpw = src_hbm.shape[0] // 32
    base = (cid * 16 + sid) * rpw
    pltpu.sync_copy(src_hbm.at[pl.ds(base, 8)], idx_s)
    pltpu.sync_copy(dst_hbm.at[pl.ds(base, 8)], idx_d)
    for j in range(8):
        pltpu.sync_copy(ob, sh_dego.at[idx_s.at[j]], add=True)
        pltpu.sync_copy(ob, sh_degi.at[idx_d.at[j]], add=True)
    plsc.subcore_barrier()
    gn = sh_dego.shape[0]
    pltpu.sync_copy(sh_dego.at[pl.ds(sid * 32, 32)],
                    degs_out.at[cid, pl.ds(sid * 32, 32)])
    pltpu.sync_copy(sh_degi.at[pl.ds(sid * 32, 32)],
                    degs_out.at[cid, pl.ds(gn + sid * 32, 32)])


def _sc_agg_body(hsrc_hbm, src_hbm, dst_hbm, agg_out, idx_s, idx_d, rows,
                 zb, sh_agg, sem):
    cid = lax.axis_index("c")
    sid = lax.axis_index("s")
    for i in range(32):
        for c in range(zb.shape[1] // 16):
            zb[i, pl.ds(16 * c, 16)] = jnp.zeros((16,), F32)
    pltpu.sync_copy(zb, sh_agg.at[pl.ds(sid * 32, 32)])
    plsc.subcore_barrier()
    rpw = src_hbm.shape[0] // 32
    base = (cid * 16 + sid) * rpw
    pltpu.sync_copy(src_hbm.at[pl.ds(base, 8)], idx_s)
    pltpu.sync_copy(dst_hbm.at[pl.ds(base, 8)], idx_d)
    for j in range(8):
        pltpu.async_copy(hsrc_hbm.at[idx_s.at[j]], rows, sem).wait()
        pltpu.sync_copy(rows, sh_agg.at[idx_d.at[j]], add=True)
    plsc.subcore_barrier()
    pltpu.sync_copy(sh_agg.at[pl.ds(sid * 32, 32)],
                    agg_out.at[cid, pl.ds(sid * 32, 32)])


def _scale_body(gfp_ref, degs_ref, hsrc_ref):
    degs = degs_ref[0] + degs_ref[1]
    rsq_o = lax.rsqrt(jnp.maximum(degs[0:512, 0:1], 1.0))
    hsrc_ref[...] = gfp_ref[...] * rsq_o


def _global_body(degs_ref, aggp_ref, tf_ref, gnf_ref, wgcn_ref, wl2a_ref,
                 wl2b_ref, wclsa_ref, wclsb_ref, bp_ref, out_ref):
    bp = bp_ref[...]
    td = wgcn_ref.shape[0]
    b_gcn = bp[0:1, 0:wgcn_ref.shape[1]]
    b_l2 = bp[1:2, 0:wl2a_ref.shape[1]]
    b_cls = bp[2:3, 0:out_ref.shape[1]]
    degs = degs_ref[0] + degs_ref[1]
    rsq_i = lax.rsqrt(jnp.maximum(degs[512:1024, 0:1], 1.0))
    agg = (aggp_ref[0] + aggp_ref[1]) * rsq_i
    gcn = jnp.maximum(_mm(agg[:, 0:td], wgcn_ref[...]) + b_gcn, 0.0)
    tra = _mm(gnf_ref[...], wl2a_ref[...]) + _mm(tf_ref[...], wl2b_ref[...]) + b_l2
    out_ref[...] = _mm(tra, wclsa_ref[...]) + _mm(gcn, wclsb_ref[...]) + b_cls


def _attn_mask(attn):
    n_heads, head_dim = attn.shape
    hd = n_heads * head_dim
    idx = jnp.arange(hd)
    return jnp.zeros((hd, n_heads), F32).at[idx, idx // head_dim].set(
        attn.reshape(-1))


def _full_spec(shape):
    nd = len(shape)
    return pl.BlockSpec(shape, lambda *_, _nd=nd: (0,) * _nd)


def kernel(sub_x, sub_edge_index, g_edge_index, g_feat, traFeat, params):
    p = params
    s, n, f_in = sub_x.shape
    e = sub_edge_index.shape[2]
    hd = p['W_src0'].shape[1]
    out_dim = p['W_lin'].shape[1]

    n_heads = p['attn0'].shape[0]
    xp = jnp.concatenate([sub_x, jnp.ones((s, n, 1), F32)],
                         axis=-1).astype(BF16)
    et = jnp.transpose(sub_edge_index.astype(jnp.int32), (0, 2, 1))
    ws0 = jnp.concatenate([p['W_src0'], p['b_src0'][None, :]],
                          axis=0).astype(BF16)
    wd0 = jnp.concatenate([p['W_dst0'], p['b_dst0'][None, :]],
                          axis=0).astype(BF16)
    wr0 = jnp.concatenate([p['res_W0'], p['res_b0'][None, :]],
                          axis=0).astype(BF16)
    am0 = _attn_mask(p['attn0']).astype(BF16)
    am1 = _attn_mask(p['attn1']).astype(BF16)
    em = (jnp.arange(hd)[None, :] // (hd // n_heads)
          == jnp.arange(n_heads)[:, None]).astype(BF16)
    bp = jnp.zeros((8, hd), F32)
    bp = bp.at[0, :].set(p['b_src1'])
    bp = bp.at[1, :].set(p['b_dst1'])
    bp = bp.at[2, 0].set(p['b_gate'][0])
    bp = bp.at[3, 0:out_dim].set(p['b_lin'])

    blk = 16
    gnf = pl.pallas_call(
        _sub_body,
        grid=(s // blk,),
        in_specs=[
            pl.BlockSpec((blk, n, f_in + 1), lambda i: (i, 0, 0)),
            pl.BlockSpec((blk, e, 2), lambda i: (i, 0, 0)),
            _full_spec(ws0.shape), _full_spec(wd0.shape),
            _full_spec(wr0.shape), _full_spec(am0.shape),
            _full_spec(p['W_src1'].shape), _full_spec(p['W_dst1'].shape),
            _full_spec(am1.shape), _full_spec(p['W_gate'].shape),
            _full_spec(p['W_lin'].shape), _full_spec(em.shape),
            _full_spec(bp.shape),
        ],
        out_specs=pl.BlockSpec((blk, 1, out_dim), lambda i: (i, 0, 0)),
        out_shape=jax.ShapeDtypeStruct((s, 1, out_dim), F32),
    )(xp, et, ws0, wd0, wr0, am0, p['W_src1'].astype(BF16),
      p['W_dst1'].astype(BF16), am1, p['W_gate'].astype(BF16),
      p['W_lin'].astype(BF16), em, bp)
    gnf = gnf.reshape(s, out_dim)

    gn, td = g_feat.shape
    ge = g_edge_index.astype(jnp.int32)
    src2d = ge[0].reshape(-1, 32)
    dst2d = ge[1].reshape(-1, 32)
    gfp = jnp.concatenate([g_feat, jnp.zeros((gn, 128 - td), F32)], axis=1)

    mesh = plsc.VectorSubcoreMesh(core_axis_name="c", subcore_axis_name="s")
    degs = pl.kernel(
        _sc_deg_body, mesh=mesh,
        out_type=jax.ShapeDtypeStruct((2, 2 * gn, 128), F32),
        scratch_types=[
            pltpu.VMEM((8, 32), jnp.int32), pltpu.VMEM((8, 32), jnp.int32),
            pltpu.VMEM((32, 128), F32), pltpu.VMEM((32, 128), F32),
            pltpu.VMEM_SHARED((gn, 128), F32),
            pltpu.VMEM_SHARED((gn, 128), F32),
            pltpu.SemaphoreType.DMA,
        ])(src2d, dst2d)

    hsrc = pl.pallas_call(
        _scale_body,
        in_specs=[_full_spec(gfp.shape), _full_spec(degs.shape)],
        out_specs=_full_spec(gfp.shape),
        out_shape=jax.ShapeDtypeStruct(gfp.shape, F32),
    )(gfp, degs)

    aggp = pl.kernel(
        _sc_agg_body, mesh=mesh,
        out_type=jax.ShapeDtypeStruct((2, gn, 128), F32),
        scratch_types=[
            pltpu.VMEM((8, 32), jnp.int32), pltpu.VMEM((8, 32), jnp.int32),
            pltpu.VMEM((32, 128), F32), pltpu.VMEM((32, 128), F32),
            pltpu.VMEM_SHARED((gn, 128), F32),
            pltpu.SemaphoreType.DMA,
        ])(hsrc, src2d, dst2d)

    wl2a = p['W_l2'][:out_dim, :]
    wl2b = p['W_l2'][out_dim:, :]
    h_dim = wl2a.shape[1]
    wclsa = p['W_cls'][:h_dim, :]
    wclsb = p['W_cls'][h_dim:, :]
    bp2 = jnp.zeros((4, max(td, h_dim)), F32)
    bp2 = bp2.at[0, 0:td].set(p['b_gcn'])
    bp2 = bp2.at[1, 0:h_dim].set(p['b_l2'])
    bp2 = bp2.at[2, 0:2].set(p['b_cls'])

    out = pl.pallas_call(
        _global_body,
        in_specs=[_full_spec(degs.shape), _full_spec(aggp.shape),
                  _full_spec(traFeat.shape), _full_spec(gnf.shape),
                  _full_spec(p['W_gcn'].shape), _full_spec(wl2a.shape),
                  _full_spec(wl2b.shape), _full_spec(wclsa.shape),
                  _full_spec(wclsb.shape), _full_spec(bp2.shape)],
        out_specs=_full_spec((gn, 2)),
        out_shape=jax.ShapeDtypeStruct((gn, 2), F32),
    )(degs, aggp, traFeat, gnf, p['W_gcn'], wl2a, wl2b, wclsa, wclsb, bp2)
    return out
